# baseline (speedup 1.0000x reference)
import functools

import jax
import jax.numpy as jnp
import numpy as np
from jax.experimental import pallas as pl
from jax.experimental.pallas import tpu as pltpu

_VMEM_LIMIT = 100 * 1024 * 1024
_HIGHEST = jax.lax.Precision.HIGHEST


def _ru(x, m):
    return ((x + m - 1) // m) * m


def _bilinear_matrix(out, inn):
    c = (np.arange(out, dtype=np.float64) + 0.5) * (inn / out) - 0.5
    c = np.clip(c, 0.0, inn - 1)
    lo = np.floor(c).astype(np.int32)
    hi = np.minimum(lo + 1, inn - 1)
    f = (c - lo).astype(np.float32)
    R = np.zeros((out, inn), np.float32)
    R[np.arange(out), lo] += 1.0 - f
    R[np.arange(out), hi] += f
    return R


def _up_pad_matrix(rows, stride, H, W, Q):
    R = np.zeros((rows, Q), np.float32)
    W2 = W + 2
    for oy in range(H):
        for ox in range(W):
            R[(oy // 2) * stride + (ox // 2), (oy + 1) * W2 + (ox + 1)] = 1.0
    return R


def _dot_t(w, a):
    return jax.lax.dot_general(w, a, (((0,), (0,)), ((), ())),
                               preferred_element_type=jnp.float32)


def _conv_t(src, w, bcol, s_offs, P):
    a = jnp.concatenate([src[:, off:off + P] for off in s_offs], axis=0)
    return jnp.maximum(_dot_t(w, a) + bcol, 0.0)


def _embed_t(x, mask, lead, Q, P, dtype):
    C = x.shape[0]
    xm = (x * mask).astype(dtype)
    return jnp.concatenate(
        [jnp.zeros((C, lead), dtype), xm,
         jnp.zeros((C, Q - P - lead), dtype)], axis=1)


def _att_alpha(x1, x2, wa_ref, ba_ref, mask, s_offs, lead, Q, P):
    att = jnp.concatenate(
        [jnp.mean(x1, axis=0, keepdims=True),
         jnp.max(x1, axis=0, keepdims=True),
         jnp.mean(x2, axis=0, keepdims=True),
         jnp.max(x2, axis=0, keepdims=True)], axis=0)
    att_full = _embed_t(att, mask, lead, Q, P, jnp.float32)
    acc = jnp.zeros((4, P), jnp.float32)
    for s, off in enumerate(s_offs):
        acc = acc + att_full[:, off:off + P] * wa_ref[s]
    a1 = jnp.sum(acc, axis=0, keepdims=True) + ba_ref[...]
    return jax.nn.sigmoid(a1)


def _seg_body(x5_ref, x4_ref, x3_ref,
              pw_ref, ow_ref, w4_ref, wa4_ref, ba4_ref,
              w3_ref, wa3_ref, ba3_ref, wc_ref, wp_ref, bp_ref,
              sel_ref, uT_ref, r4_ref, r3_ref,
              o_ref,
              *, sizes, cin5, cin4, cin3, C, ncls):
    bp = bp_ref[...]

    aT = jnp.dot(x5_ref[...], sel_ref[...],
                 preferred_element_type=jnp.float32)
    ys = []
    r0 = 0
    for bi, ps in enumerate(sizes):
        n = ps * ps
        wb = pw_ref[bi * cin5:(bi + 1) * cin5, :]
        y = _dot_t(wb, aT[:, r0:r0 + n].astype(jnp.bfloat16))
        ys.append(jnp.maximum(y + bp[:, bi:bi + 1] + bp[:, 3:4], 0.0))
        r0 += n
    ycatT = jnp.concatenate(ys, axis=1)
    fusedT = jnp.dot(ycatT, uT_ref[...],
                     preferred_element_type=jnp.float32, precision=_HIGHEST)
    xT = _dot_t(ow_ref[...], fusedT.astype(jnp.bfloat16)) + bp[:, 4:5]
    xT = jnp.maximum(xT, 0.0)

    H4, W4 = 32, 32
    W24 = W4 + 2
    P4 = (H4 + 2) * W24
    offs4 = [dy * W24 + dx for dy in range(3) for dx in range(3)]
    lead4 = W24 + 1
    Q4 = r4_ref.shape[1]
    w4l = w4_ref[0:9 * 32].reshape(9, 32, 2 * C)[:, :cin4, :C]
    w4l = w4l.reshape(9 * cin4, C)
    w4u = w4_ref[9 * 32:].reshape(9, C, 2 * C)[:, :, C:].reshape(9 * C, C)
    up4 = jnp.dot(xT.astype(jnp.bfloat16), r4_ref[...],
                  preferred_element_type=jnp.float32).astype(jnp.bfloat16)
    q4 = jax.lax.broadcasted_iota(jnp.int32, (1, P4), 1)
    oy4 = q4 // W24
    ox4 = q4 - oy4 * W24
    mask4 = jnp.logical_and(oy4 < H4, ox4 < W4).astype(jnp.float32)
    x1 = _conv_t(x4_ref[...], w4l, bp[:, 5:6], offs4, P4)
    x2 = _conv_t(up4, w4u, bp[:, 6:7], offs4, P4)
    alpha = _att_alpha(x1, x2, wa4_ref, ba4_ref, mask4, offs4, lead4, Q4, P4)
    o4 = x1 * alpha + x2 * (1.0 - alpha)

    H3, W3 = 64, 64
    W23 = W3 + 2
    P3 = (H3 + 2) * W23
    offs3 = [dy * W23 + dx for dy in range(3) for dx in range(3)]
    lead3 = W23 + 1
    Q3 = r3_ref.shape[1]
    w3l = w3_ref[0:9 * 32].reshape(9, 32, 2 * C)[:, :cin3, :C]
    w3l = w3l.reshape(9 * cin3, C)
    w3u = w3_ref[9 * 32:].reshape(9, C, 2 * C)[:, :, C:].reshape(9 * C, C)
    up3 = jnp.dot(o4.astype(jnp.bfloat16), r3_ref[...],
                  preferred_element_type=jnp.float32).astype(jnp.bfloat16)
    q3 = jax.lax.broadcasted_iota(jnp.int32, (1, P3), 1)
    oy3 = q3 // W23
    ox3 = q3 - oy3 * W23
    mask3 = jnp.logical_and(oy3 < H3, ox3 < W3).astype(jnp.float32)
    x1 = _conv_t(x3_ref[...], w3l, bp[:, 7:8], offs3, P3)
    x2 = _conv_t(up3, w3u, bp[:, 8:9], offs3, P3)
    alpha = _att_alpha(x1, x2, wa3_ref, ba3_ref, mask3, offs3, lead3, Q3, P3)
    o3 = x1 * alpha + x2 * (1.0 - alpha)

    L = wc_ref.shape[0]
    y = o3
    for l in range(L):
        y_full = _embed_t(y, mask3, lead3, Q3, P3, jnp.bfloat16)
        y = _conv_t(y_full, wc_ref[l], bp[:, 9 + l:10 + l], offs3, P3)
    logits = _dot_t(wp_ref[...], y.astype(jnp.bfloat16)) + bp[:ncls, 11:12]
    m = jnp.max(logits, axis=0, keepdims=True)
    denom = jnp.sum(jnp.exp(logits - m), axis=0, keepdims=True)
    score = 1.0 / denom
    cidx = jax.lax.broadcasted_iota(jnp.int32, logits.shape, 0).astype(
        jnp.float32)
    cls = jnp.min(jnp.where(logits == m, cidx, float(ncls)),
                  axis=0, keepdims=True)
    pad = jnp.zeros((o_ref.shape[0] - 2, P3), jnp.float32)
    o_ref[...] = jnp.concatenate([score, cls, pad], axis=0)


def _up_body(sc_ref, e_ref, et_ref, so_ref, co_ref):
    e = e_ref[...]
    et = et_ref[...]
    s = jnp.dot(e, sc_ref[0, :64, :64], preferred_element_type=jnp.float32,
                precision=_HIGHEST)
    so_ref[...] = jnp.dot(s, et, preferred_element_type=jnp.float32,
                          precision=_HIGHEST)
    c = jnp.dot(e, sc_ref[1, :64, :64], preferred_element_type=jnp.float32)
    co_ref[...] = jnp.dot(c, et, preferred_element_type=jnp.float32).astype(
        jnp.int32)


def _upsample(sc, r):
    B = sc.shape[0]
    H = 64
    E = np.zeros((H * r, H), np.float32)
    E[np.arange(H * r), np.arange(H * r) // r] = 1.0
    Ej = jnp.asarray(E)
    return pl.pallas_call(
        _up_body,
        out_shape=(jax.ShapeDtypeStruct((B, H * r, H * r), jnp.float32),
                   jax.ShapeDtypeStruct((B, H * r, H * r), jnp.int32)),
        grid=(B,),
        in_specs=[
            pl.BlockSpec((None, 2, 66, 66), lambda b: (b, 0, 0, 0)),
            pl.BlockSpec((H * r, H), lambda b: (0, 0)),
            pl.BlockSpec((H, H * r), lambda b: (0, 0)),
        ],
        out_specs=(pl.BlockSpec((None, H * r, H * r), lambda b: (b, 0, 0)),
                   pl.BlockSpec((None, H * r, H * r), lambda b: (b, 0, 0))),
        compiler_params=pltpu.CompilerParams(
            dimension_semantics=("parallel",), vmem_limit_bytes=_VMEM_LIMIT),
    )(sc, Ej, Ej.T)


def _pad_flat_t(x, q):
    B, C, H, W = x.shape
    xp = jnp.pad(x.astype(jnp.bfloat16), ((0, 0), (0, 0), (1, 1), (1, 1)))
    flat = xp.reshape(B, C, (H + 2) * (W + 2))
    return jnp.pad(flat, ((0, 0), (0, 0), (0, q - flat.shape[-1])))


def kernel(sppm_pool_w, sppm_pool_b, sppm_out_w, sppm_out_b,
           out_w_blocks, out_b_blocks, out_w_proj, out_b_proj,
           level0_w, level0_b, level0_w_att, level0_b_att,
           level1_w, level1_b, level1_w_att, level1_b_att,
           input_0, input_1, input_2, input_3, input_4, input_5):
    B = input_5.shape[0]
    C = 32
    sizes = (1, 2, 4)
    Q4 = _ru(1156 + 2 * 34 + 2, 128)
    Q3 = _ru(4356 + 2 * 66 + 2, 128)

    x5r = input_5.reshape(B, C, 256).astype(jnp.bfloat16)
    x4p = _pad_flat_t(input_4, Q4)
    x3p = _pad_flat_t(input_3, Q3)

    sel = np.zeros((256, 21), np.float32)
    col = 0
    for ps in sizes:
        st = 16 // ps
        for yy in range(ps):
            for xx in range(ps):
                sel[(yy * st) * 16 + xx * st, col] = 1.0
                col += 1
    U = np.concatenate(
        [np.kron(_bilinear_matrix(16, ps), _bilinear_matrix(16, ps))
         for ps in sizes], axis=1)
    uT = jnp.asarray(U.T, jnp.float32)
    selj = jnp.asarray(sel, jnp.bfloat16)
    r4 = jnp.asarray(_up_pad_matrix(256, 16, 32, 32, Q4), jnp.bfloat16)
    r3 = jnp.asarray(_up_pad_matrix(1156, 34, 64, 64, Q3), jnp.bfloat16)

    bpack = jnp.concatenate(
        [sppm_pool_w[3 * C:3 * C + 3].astype(jnp.float32),
         sppm_pool_b, sppm_out_b,
         level0_b.reshape(2, C), level1_b.reshape(2, C),
         out_b_blocks.reshape(2, C),
         jnp.pad(out_b_proj, ((0, 0), (0, C - out_b_proj.shape[1])))],
        axis=0).T

    wa4 = jnp.transpose(level0_w_att, (0, 2, 1))
    wa3 = jnp.transpose(level1_w_att, (0, 2, 1))

    consts = (sppm_pool_w, sppm_out_w, level0_w, wa4, level0_b_att,
              level1_w, wa3, level1_b_att, out_w_blocks, out_w_proj, bpack,
              selj, uT, r4, r3)
    P3 = 66 * 66
    specs = [pl.BlockSpec((None,) + x.shape[1:], lambda b: (b, 0, 0))
             for x in (x5r, x4p, x3p)]
    specs += [pl.BlockSpec(c.shape, lambda b, n=c.ndim: (0,) * n)
              for c in consts]
    out = pl.pallas_call(
        functools.partial(_seg_body, sizes=sizes, cin5=32, cin4=24,
                          cin3=16, C=C, ncls=5),
        out_shape=jax.ShapeDtypeStruct((B, 8, P3), jnp.float32),
        grid=(B,),
        in_specs=specs,
        out_specs=pl.BlockSpec((None, 8, P3), lambda b: (b, 0, 0)),
        compiler_params=pltpu.CompilerParams(
            dimension_semantics=("parallel",), vmem_limit_bytes=_VMEM_LIMIT),
    )(x5r, x4p, x3p, *consts)

    sc = out.reshape(B, 8, 66, 66)
    H0 = input_0.shape[2]
    return _upsample(sc, H0 // 64)


# speedup vs baseline: 1.1446x; 1.1446x over previous
import functools

import jax
import jax.numpy as jnp
import numpy as np
from jax.experimental import pallas as pl
from jax.experimental.pallas import tpu as pltpu

_VMEM_LIMIT = 100 * 1024 * 1024
_HIGHEST = jax.lax.Precision.HIGHEST


def _ru(x, m):
    return ((x + m - 1) // m) * m


def _bilinear_matrix(out, inn):
    c = (np.arange(out, dtype=np.float64) + 0.5) * (inn / out) - 0.5
    c = np.clip(c, 0.0, inn - 1)
    lo = np.floor(c).astype(np.int32)
    hi = np.minimum(lo + 1, inn - 1)
    f = (c - lo).astype(np.float32)
    R = np.zeros((out, inn), np.float32)
    R[np.arange(out), lo] += 1.0 - f
    R[np.arange(out), hi] += f
    return R


def _up_pad_matrix(rows, stride, H, W, Q):
    R = np.zeros((rows, Q), np.float32)
    W2 = W + 2
    for oy in range(H):
        for ox in range(W):
            R[(oy // 2) * stride + (ox // 2), (oy + 1) * W2 + (ox + 1)] = 1.0
    return R


def _dot_t(w, a):
    return jax.lax.dot_general(w, a, (((0,), (0,)), ((), ())),
                               preferred_element_type=jnp.float32)


def _conv_t(src, w, bcol, s_offs, P):
    a = jnp.concatenate([src[:, off:off + P] for off in s_offs], axis=0)
    return jnp.maximum(_dot_t(w, a) + bcol, 0.0)


def _embed_t(x, mask, lead, Q, P, dtype):
    C = x.shape[0]
    xm = (x * mask).astype(dtype)
    return jnp.concatenate(
        [jnp.zeros((C, lead), dtype), xm,
         jnp.zeros((C, Q - P - lead), dtype)], axis=1)


def _att_alpha(x1, x2, wa_ref, ba_ref, mask, s_offs, lead, Q, P):
    att = jnp.concatenate(
        [jnp.mean(x1, axis=0, keepdims=True),
         jnp.max(x1, axis=0, keepdims=True),
         jnp.mean(x2, axis=0, keepdims=True),
         jnp.max(x2, axis=0, keepdims=True)], axis=0)
    att_full = _embed_t(att, mask, lead, Q, P, jnp.float32)
    acc = jnp.zeros((4, P), jnp.float32)
    for s, off in enumerate(s_offs):
        acc = acc + att_full[:, off:off + P] * wa_ref[s]
    a1 = jnp.sum(acc, axis=0, keepdims=True) + ba_ref[...]
    return jax.nn.sigmoid(a1)


def _seg_body(x5_ref, x4_ref, x3_ref,
              pw_ref, ow_ref, w4_ref, wa4_ref, ba4_ref,
              w3_ref, wa3_ref, ba3_ref, wc_ref, wp_ref, bp_ref,
              sel_ref, uT_ref, r4_ref, r3_ref,
              o_ref,
              *, sizes, cin5, cin4, cin3, C, ncls, NB):
    bp = bp_ref[...]

    def sppm_one(i):
        aT = jnp.dot(x5_ref[i], sel_ref[...],
                     preferred_element_type=jnp.float32)
        ys = []
        r0 = 0
        for bi, ps in enumerate(sizes):
            n = ps * ps
            wb = pw_ref[bi * cin5:(bi + 1) * cin5, :]
            y = _dot_t(wb, aT[:, r0:r0 + n].astype(jnp.bfloat16))
            ys.append(jnp.maximum(y + bp[:, bi:bi + 1] + bp[:, 3:4], 0.0))
            r0 += n
        ycatT = jnp.concatenate(ys, axis=1)
        fusedT = jnp.dot(ycatT, uT_ref[...],
                         preferred_element_type=jnp.float32,
                         precision=_HIGHEST)
        xT = _dot_t(ow_ref[...], fusedT.astype(jnp.bfloat16)) + bp[:, 4:5]
        return jnp.maximum(xT, 0.0)

    xTs = jnp.concatenate([sppm_one(i).astype(jnp.bfloat16)
                           for i in range(NB)], axis=0)

    H4, W4 = 32, 32
    W24 = W4 + 2
    P4 = (H4 + 2) * W24
    offs4 = [dy * W24 + dx for dy in range(3) for dx in range(3)]
    lead4 = W24 + 1
    w4l = w4_ref[0:9 * 32].reshape(9, 32, 2 * C)[:, :cin4, :C]
    w4l = w4l.reshape(9 * cin4, C)
    w4u = w4_ref[9 * 32:].reshape(9, C, 2 * C)[:, :, C:].reshape(9 * C, C)
    up4s = jnp.dot(xTs, r4_ref[...],
                   preferred_element_type=jnp.float32).astype(jnp.bfloat16)
    q4 = jax.lax.broadcasted_iota(jnp.int32, (1, P4), 1)
    oy4 = q4 // W24
    ox4 = q4 - oy4 * W24
    mask4 = jnp.logical_and(oy4 < H4, ox4 < W4).astype(jnp.float32)

    def level4_one(i):
        x1 = _conv_t(x4_ref[i], w4l, bp[:, 5:6], offs4, P4)
        x2 = _conv_t(up4s[i * C:(i + 1) * C], w4u, bp[:, 6:7], offs4, P4)
        alpha = _att_alpha(x1, x2, wa4_ref, ba4_ref, mask4, offs4, lead4,
                           r4_ref.shape[1], P4)
        o4 = x1 * alpha + x2 * (1.0 - alpha)
        return o4.astype(jnp.bfloat16)

    o4s = jnp.concatenate([level4_one(i) for i in range(NB)], axis=0)

    H3, W3 = 64, 64
    W23 = W3 + 2
    P3 = (H3 + 2) * W23
    offs3 = [dy * W23 + dx for dy in range(3) for dx in range(3)]
    lead3 = W23 + 1
    Q3 = r3_ref.shape[1]
    w3l = w3_ref[0:9 * 32].reshape(9, 32, 2 * C)[:, :cin3, :C]
    w3l = w3l.reshape(9 * cin3, C)
    w3u = w3_ref[9 * 32:].reshape(9, C, 2 * C)[:, :, C:].reshape(9 * C, C)
    up3s = jnp.dot(o4s, r3_ref[...],
                   preferred_element_type=jnp.float32).astype(jnp.bfloat16)
    q3 = jax.lax.broadcasted_iota(jnp.int32, (1, P3), 1)
    oy3 = q3 // W23
    ox3 = q3 - oy3 * W23
    mask3 = jnp.logical_and(oy3 < H3, ox3 < W3).astype(jnp.float32)

    L = wc_ref.shape[0]
    for i in range(NB):
        x1 = _conv_t(x3_ref[i], w3l, bp[:, 7:8], offs3, P3)
        x2 = _conv_t(up3s[i * C:(i + 1) * C], w3u, bp[:, 8:9], offs3, P3)
        alpha = _att_alpha(x1, x2, wa3_ref, ba3_ref, mask3, offs3, lead3,
                           Q3, P3)
        y = x1 * alpha + x2 * (1.0 - alpha)

        for l in range(L):
            y_full = _embed_t(y, mask3, lead3, Q3, P3, jnp.bfloat16)
            y = _conv_t(y_full, wc_ref[l], bp[:, 9 + l:10 + l], offs3, P3)
        logits = _dot_t(wp_ref[...], y.astype(jnp.bfloat16)) + bp[:ncls,
                                                                  11:12]
        m = jnp.max(logits, axis=0, keepdims=True)
        denom = jnp.sum(jnp.exp(logits - m), axis=0, keepdims=True)
        score = 1.0 / denom
        cidx = jax.lax.broadcasted_iota(jnp.int32, logits.shape, 0).astype(
            jnp.float32)
        cls = jnp.min(jnp.where(logits == m, cidx, float(ncls)),
                      axis=0, keepdims=True)
        pad = jnp.zeros((o_ref.shape[1] - 2, P3), jnp.float32)
        o_ref[i] = jnp.concatenate([score, cls, pad], axis=0)


def _up_body(sc_ref, e_ref, et_ref, so_ref, co_ref):
    e = e_ref[...]
    et = et_ref[...]
    s = jnp.dot(e, sc_ref[0, :64, :64], preferred_element_type=jnp.float32,
                precision=_HIGHEST)
    so_ref[...] = jnp.dot(s, et, preferred_element_type=jnp.float32,
                          precision=_HIGHEST)
    c = jnp.dot(e, sc_ref[1, :64, :64], preferred_element_type=jnp.float32)
    co_ref[...] = jnp.dot(c, et, preferred_element_type=jnp.float32).astype(
        jnp.int32)


def _upsample(sc, r):
    B = sc.shape[0]
    H = 64
    E = np.zeros((H * r, H), np.float32)
    E[np.arange(H * r), np.arange(H * r) // r] = 1.0
    Ej = jnp.asarray(E)
    return pl.pallas_call(
        _up_body,
        out_shape=(jax.ShapeDtypeStruct((B, H * r, H * r), jnp.float32),
                   jax.ShapeDtypeStruct((B, H * r, H * r), jnp.int32)),
        grid=(B,),
        in_specs=[
            pl.BlockSpec((None, 2, 66, 66), lambda b: (b, 0, 0, 0)),
            pl.BlockSpec((H * r, H), lambda b: (0, 0)),
            pl.BlockSpec((H, H * r), lambda b: (0, 0)),
        ],
        out_specs=(pl.BlockSpec((None, H * r, H * r), lambda b: (b, 0, 0)),
                   pl.BlockSpec((None, H * r, H * r), lambda b: (b, 0, 0))),
        compiler_params=pltpu.CompilerParams(
            dimension_semantics=("parallel",), vmem_limit_bytes=_VMEM_LIMIT),
    )(sc, Ej, Ej.T)


def _pad_flat_t(x, q):
    B, C, H, W = x.shape
    xp = jnp.pad(x.astype(jnp.bfloat16), ((0, 0), (0, 0), (1, 1), (1, 1)))
    flat = xp.reshape(B, C, (H + 2) * (W + 2))
    return jnp.pad(flat, ((0, 0), (0, 0), (0, q - flat.shape[-1])))


def kernel(sppm_pool_w, sppm_pool_b, sppm_out_w, sppm_out_b,
           out_w_blocks, out_b_blocks, out_w_proj, out_b_proj,
           level0_w, level0_b, level0_w_att, level0_b_att,
           level1_w, level1_b, level1_w_att, level1_b_att,
           input_0, input_1, input_2, input_3, input_4, input_5):
    B = input_5.shape[0]
    C = 32
    sizes = (1, 2, 4)
    Q4 = _ru(1156 + 2 * 34 + 2, 128)
    Q3 = _ru(4356 + 2 * 66 + 2, 128)

    x5r = input_5.reshape(B, C, 256).astype(jnp.bfloat16)
    x4p = _pad_flat_t(input_4, Q4)
    x3p = _pad_flat_t(input_3, Q3)

    sel = np.zeros((256, 21), np.float32)
    col = 0
    for ps in sizes:
        st = 16 // ps
        for yy in range(ps):
            for xx in range(ps):
                sel[(yy * st) * 16 + xx * st, col] = 1.0
                col += 1
    U = np.concatenate(
        [np.kron(_bilinear_matrix(16, ps), _bilinear_matrix(16, ps))
         for ps in sizes], axis=1)
    uT = jnp.asarray(U.T, jnp.float32)
    selj = jnp.asarray(sel, jnp.bfloat16)
    r4 = jnp.asarray(_up_pad_matrix(256, 16, 32, 32, Q4), jnp.bfloat16)
    r3 = jnp.asarray(_up_pad_matrix(1156, 34, 64, 64, Q3), jnp.bfloat16)

    bpack = jnp.concatenate(
        [sppm_pool_w[3 * C:3 * C + 3].astype(jnp.float32),
         sppm_pool_b, sppm_out_b,
         level0_b.reshape(2, C), level1_b.reshape(2, C),
         out_b_blocks.reshape(2, C),
         jnp.pad(out_b_proj, ((0, 0), (0, C - out_b_proj.shape[1])))],
        axis=0).T

    wa4 = jnp.transpose(level0_w_att, (0, 2, 1))
    wa3 = jnp.transpose(level1_w_att, (0, 2, 1))

    consts = (sppm_pool_w, sppm_out_w, level0_w, wa4, level0_b_att,
              level1_w, wa3, level1_b_att, out_w_blocks, out_w_proj, bpack,
              selj, uT, r4, r3)
    P3 = 66 * 66
    NB = 4
    specs = [pl.BlockSpec((NB,) + x.shape[1:], lambda b: (b, 0, 0))
             for x in (x5r, x4p, x3p)]
    specs += [pl.BlockSpec(c.shape, lambda b, n=c.ndim: (0,) * n)
              for c in consts]
    out = pl.pallas_call(
        functools.partial(_seg_body, sizes=sizes, cin5=32, cin4=24,
                          cin3=16, C=C, ncls=5, NB=NB),
        out_shape=jax.ShapeDtypeStruct((B, 8, P3), jnp.float32),
        grid=(B // NB,),
        in_specs=specs,
        out_specs=pl.BlockSpec((NB, 8, P3), lambda b: (b, 0, 0)),
        compiler_params=pltpu.CompilerParams(
            dimension_semantics=("parallel",), vmem_limit_bytes=_VMEM_LIMIT),
    )(x5r, x4p, x3p, *consts)

    sc = out.reshape(B, 8, 66, 66)
    H0 = input_0.shape[2]
    return _upsample(sc, H0 // 64)


# speedup vs baseline: 1.1630x; 1.0161x over previous
import functools

import jax
import jax.numpy as jnp
import numpy as np
from jax.experimental import pallas as pl
from jax.experimental.pallas import tpu as pltpu

_VMEM_LIMIT = 100 * 1024 * 1024
_HIGHEST = jax.lax.Precision.HIGHEST


def _ru(x, m):
    return ((x + m - 1) // m) * m


def _bilinear_matrix(out, inn):
    c = (np.arange(out, dtype=np.float64) + 0.5) * (inn / out) - 0.5
    c = np.clip(c, 0.0, inn - 1)
    lo = np.floor(c).astype(np.int32)
    hi = np.minimum(lo + 1, inn - 1)
    f = (c - lo).astype(np.float32)
    R = np.zeros((out, inn), np.float32)
    R[np.arange(out), lo] += 1.0 - f
    R[np.arange(out), hi] += f
    return R


def _up_pad_matrix(rows, stride, H, W, Q):
    R = np.zeros((rows, Q), np.float32)
    W2 = W + 2
    for oy in range(H):
        for ox in range(W):
            R[(oy // 2) * stride + (ox // 2), (oy + 1) * W2 + (ox + 1)] = 1.0
    return R


def _dot_t(w, a):
    return jax.lax.dot_general(w, a, (((0,), (0,)), ((), ())),
                               preferred_element_type=jnp.float32)


def _conv_t(src, w, bcol, s_offs, P):
    a = jnp.concatenate([src[:, off:off + P] for off in s_offs], axis=0)
    return jnp.maximum(_dot_t(w, a) + bcol, 0.0)


def _embed_t(x, mask, lead, Q, P, dtype):
    C = x.shape[0]
    xm = (x * mask).astype(dtype)
    return jnp.concatenate(
        [jnp.zeros((C, lead), dtype), xm,
         jnp.zeros((C, Q - P - lead), dtype)], axis=1)


def _att_alpha(x1, x2, wa_ref, ba_ref, mask, s_offs, lead, Q, P):
    att = jnp.concatenate(
        [jnp.mean(x1, axis=0, keepdims=True),
         jnp.max(x1, axis=0, keepdims=True),
         jnp.mean(x2, axis=0, keepdims=True),
         jnp.max(x2, axis=0, keepdims=True)], axis=0)
    att_full = _embed_t(att, mask, lead, Q, P, jnp.float32)
    acc = jnp.zeros((4, P), jnp.float32)
    for s, off in enumerate(s_offs):
        acc = acc + att_full[:, off:off + P] * wa_ref[s]
    a1 = jnp.sum(acc, axis=0, keepdims=True) + ba_ref[...]
    return jax.nn.sigmoid(a1)


def _seg_body(x5_ref, x4_ref, x3_ref,
              pw_ref, ow_ref, w4_ref, wa4_ref, ba4_ref,
              w3_ref, wa3_ref, ba3_ref, wc_ref, wp_ref, bp_ref,
              sel_ref, uT_ref, r4_ref, r3_ref,
              o_ref,
              *, sizes, cin5, cin4, cin3, C, ncls, NB):
    bp = bp_ref[...]

    def sppm_one(i):
        aT = jnp.dot(x5_ref[i], sel_ref[...],
                     preferred_element_type=jnp.float32)
        ys = []
        r0 = 0
        for bi, ps in enumerate(sizes):
            n = ps * ps
            wb = pw_ref[bi * cin5:(bi + 1) * cin5, :]
            y = _dot_t(wb, aT[:, r0:r0 + n].astype(jnp.bfloat16))
            ys.append(jnp.maximum(y + bp[:, bi:bi + 1] + bp[:, 3:4], 0.0))
            r0 += n
        ycatT = jnp.concatenate(ys, axis=1)
        fusedT = jnp.dot(ycatT, uT_ref[...],
                         preferred_element_type=jnp.float32,
                         precision=_HIGHEST)
        xT = _dot_t(ow_ref[...], fusedT.astype(jnp.bfloat16)) + bp[:, 4:5]
        return jnp.maximum(xT, 0.0)

    xTs = jnp.concatenate([sppm_one(i).astype(jnp.bfloat16)
                           for i in range(NB)], axis=0)

    H4, W4 = 32, 32
    W24 = W4 + 2
    P4 = (H4 + 2) * W24
    offs4 = [dy * W24 + dx for dy in range(3) for dx in range(3)]
    lead4 = W24 + 1
    w4l = w4_ref[0:9 * 32].reshape(9, 32, 2 * C)[:, :cin4, :C]
    w4l = w4l.reshape(9 * cin4, C)
    w4u = w4_ref[9 * 32:].reshape(9, C, 2 * C)[:, :, C:].reshape(9 * C, C)
    up4s = jnp.dot(xTs, r4_ref[...],
                   preferred_element_type=jnp.float32).astype(jnp.bfloat16)
    q4 = jax.lax.broadcasted_iota(jnp.int32, (1, P4), 1)
    oy4 = q4 // W24
    ox4 = q4 - oy4 * W24
    mask4 = jnp.logical_and(oy4 < H4, ox4 < W4).astype(jnp.float32)

    def level4_one(i):
        x1 = _conv_t(x4_ref[i], w4l, bp[:, 5:6], offs4, P4)
        x2 = _conv_t(up4s[i * C:(i + 1) * C], w4u, bp[:, 6:7], offs4, P4)
        alpha = _att_alpha(x1, x2, wa4_ref, ba4_ref, mask4, offs4, lead4,
                           r4_ref.shape[1], P4)
        o4 = x1 * alpha + x2 * (1.0 - alpha)
        return o4.astype(jnp.bfloat16)

    o4s = jnp.concatenate([level4_one(i) for i in range(NB)], axis=0)

    H3, W3 = 64, 64
    W23 = W3 + 2
    P3 = (H3 + 2) * W23
    offs3 = [dy * W23 + dx for dy in range(3) for dx in range(3)]
    lead3 = W23 + 1
    Q3 = r3_ref.shape[1]
    w3l = w3_ref[0:9 * 32].reshape(9, 32, 2 * C)[:, :cin3, :C]
    w3l = w3l.reshape(9 * cin3, C)
    w3u = w3_ref[9 * 32:].reshape(9, C, 2 * C)[:, :, C:].reshape(9 * C, C)
    up3s = jnp.dot(o4s, r3_ref[...],
                   preferred_element_type=jnp.float32).astype(jnp.bfloat16)
    q3 = jax.lax.broadcasted_iota(jnp.int32, (1, P3), 1)
    oy3 = q3 // W23
    ox3 = q3 - oy3 * W23
    mask3 = jnp.logical_and(oy3 < H3, ox3 < W3).astype(jnp.float32)

    L = wc_ref.shape[0]
    for i in range(NB):
        x1 = _conv_t(x3_ref[i], w3l, bp[:, 7:8], offs3, P3)
        x2 = _conv_t(up3s[i * C:(i + 1) * C], w3u, bp[:, 8:9], offs3, P3)
        alpha = _att_alpha(x1, x2, wa3_ref, ba3_ref, mask3, offs3, lead3,
                           Q3, P3)
        y = x1 * alpha + x2 * (1.0 - alpha)

        for l in range(L):
            y_full = _embed_t(y, mask3, lead3, Q3, P3, jnp.bfloat16)
            y = _conv_t(y_full, wc_ref[l], bp[:, 9 + l:10 + l], offs3, P3)
        logits = _dot_t(wp_ref[...], y.astype(jnp.bfloat16)) + bp[:ncls,
                                                                  11:12]
        m = jnp.max(logits, axis=0, keepdims=True)
        denom = jnp.sum(jnp.exp(logits - m), axis=0, keepdims=True)
        score = 1.0 / denom
        cidx = jax.lax.broadcasted_iota(jnp.int32, logits.shape, 0).astype(
            jnp.float32)
        cls = jnp.min(jnp.where(logits == m, cidx, float(ncls)),
                      axis=0, keepdims=True)
        o_ref[i] = jnp.concatenate([score, cls], axis=0)


def _up_body(sc_ref, e_ref, et_ref, so_ref, co_ref):
    e = e_ref[...]
    et = et_ref[...]
    s = jnp.dot(e, sc_ref[0, :64, :64], preferred_element_type=jnp.float32,
                precision=_HIGHEST)
    so_ref[...] = jnp.dot(s, et, preferred_element_type=jnp.float32,
                          precision=_HIGHEST)
    c = jnp.dot(e, sc_ref[1, :64, :64], preferred_element_type=jnp.float32)
    co_ref[...] = jnp.dot(c, et, preferred_element_type=jnp.float32).astype(
        jnp.int32)


def _upsample(sc, r):
    B = sc.shape[0]
    H = 64
    E = np.zeros((H * r, H), np.float32)
    E[np.arange(H * r), np.arange(H * r) // r] = 1.0
    Ej = jnp.asarray(E)
    return pl.pallas_call(
        _up_body,
        out_shape=(jax.ShapeDtypeStruct((B, H * r, H * r), jnp.float32),
                   jax.ShapeDtypeStruct((B, H * r, H * r), jnp.int32)),
        grid=(B,),
        in_specs=[
            pl.BlockSpec((None, 2, 66, 66), lambda b: (b, 0, 0, 0)),
            pl.BlockSpec((H * r, H), lambda b: (0, 0)),
            pl.BlockSpec((H, H * r), lambda b: (0, 0)),
        ],
        out_specs=(pl.BlockSpec((None, H * r, H * r), lambda b: (b, 0, 0)),
                   pl.BlockSpec((None, H * r, H * r), lambda b: (b, 0, 0))),
        compiler_params=pltpu.CompilerParams(
            dimension_semantics=("parallel",), vmem_limit_bytes=_VMEM_LIMIT),
    )(sc, Ej, Ej.T)


def _pad_flat_t(x, q):
    B, C, H, W = x.shape
    xp = jnp.pad(x.astype(jnp.bfloat16), ((0, 0), (0, 0), (1, 1), (1, 1)))
    flat = xp.reshape(B, C, (H + 2) * (W + 2))
    return jnp.pad(flat, ((0, 0), (0, 0), (0, q - flat.shape[-1])))


def kernel(sppm_pool_w, sppm_pool_b, sppm_out_w, sppm_out_b,
           out_w_blocks, out_b_blocks, out_w_proj, out_b_proj,
           level0_w, level0_b, level0_w_att, level0_b_att,
           level1_w, level1_b, level1_w_att, level1_b_att,
           input_0, input_1, input_2, input_3, input_4, input_5):
    B = input_5.shape[0]
    C = 32
    sizes = (1, 2, 4)
    Q4 = _ru(1156 + 2 * 34 + 2, 128)
    Q3 = _ru(4356 + 2 * 66 + 2, 128)

    x5r = input_5.reshape(B, C, 256).astype(jnp.bfloat16)
    x4p = _pad_flat_t(input_4, Q4)
    x3p = _pad_flat_t(input_3, Q3)

    sel = np.zeros((256, 21), np.float32)
    col = 0
    for ps in sizes:
        st = 16 // ps
        for yy in range(ps):
            for xx in range(ps):
                sel[(yy * st) * 16 + xx * st, col] = 1.0
                col += 1
    U = np.concatenate(
        [np.kron(_bilinear_matrix(16, ps), _bilinear_matrix(16, ps))
         for ps in sizes], axis=1)
    uT = jnp.asarray(U.T, jnp.float32)
    selj = jnp.asarray(sel, jnp.bfloat16)
    r4 = jnp.asarray(_up_pad_matrix(256, 16, 32, 32, Q4), jnp.bfloat16)
    r3 = jnp.asarray(_up_pad_matrix(1156, 34, 64, 64, Q3), jnp.bfloat16)

    bpack = jnp.concatenate(
        [sppm_pool_w[3 * C:3 * C + 3].astype(jnp.float32),
         sppm_pool_b, sppm_out_b,
         level0_b.reshape(2, C), level1_b.reshape(2, C),
         out_b_blocks.reshape(2, C),
         jnp.pad(out_b_proj, ((0, 0), (0, C - out_b_proj.shape[1])))],
        axis=0).T

    wa4 = jnp.transpose(level0_w_att, (0, 2, 1))
    wa3 = jnp.transpose(level1_w_att, (0, 2, 1))

    consts = (sppm_pool_w, sppm_out_w, level0_w, wa4, level0_b_att,
              level1_w, wa3, level1_b_att, out_w_blocks, out_w_proj, bpack,
              selj, uT, r4, r3)
    P3 = 66 * 66
    NB = 4
    specs = [pl.BlockSpec((NB,) + x.shape[1:], lambda b: (b, 0, 0))
             for x in (x5r, x4p, x3p)]
    specs += [pl.BlockSpec(c.shape, lambda b, n=c.ndim: (0,) * n)
              for c in consts]
    out = pl.pallas_call(
        functools.partial(_seg_body, sizes=sizes, cin5=32, cin4=24,
                          cin3=16, C=C, ncls=5, NB=NB),
        out_shape=jax.ShapeDtypeStruct((B, 2, P3), jnp.float32),
        grid=(B // NB,),
        in_specs=specs,
        out_specs=pl.BlockSpec((NB, 2, P3), lambda b: (b, 0, 0)),
        compiler_params=pltpu.CompilerParams(
            dimension_semantics=("parallel",), vmem_limit_bytes=_VMEM_LIMIT),
    )(x5r, x4p, x3p, *consts)

    sc = out.reshape(B, 2, 66, 66)
    H0 = input_0.shape[2]
    return _upsample(sc, H0 // 64)
